# baseline (speedup 1.0000x reference)
import functools

import jax
import jax.numpy as jnp
from jax.experimental import pallas as pl
from jax.experimental.pallas import tpu as pltpu
from jax.experimental.pallas import tpu_sc as plsc

_B = 4096
_D = 1024
_C = 1000
_CP = 1024
_K = 32
_RB = 512
_IDX_BITS = 12
_IDX_MASK = (1 << _IDX_BITS) - 1
_NW = 32
_RPW = _B // _NW


def _normalize_kernel(f_ref, out_ref):
    f = f_ref[...]
    n2 = jnp.sum(f * f, axis=1, keepdims=True)
    out_ref[...] = f * jax.lax.rsqrt(jnp.maximum(n2, 1e-24))


def _simtopk_kernel(fb_ref, fn_ref, tv_ref, ti_ref):
    sim = jax.lax.dot_general(
        fb_ref[...], fn_ref[...], (((1,), (1,)), ((), ())),
        preferred_element_type=jnp.float32)
    col = jax.lax.broadcasted_iota(jnp.int32, sim.shape, 1)
    u = jax.lax.bitcast_convert_type(sim, jnp.int32)
    key = jnp.where(u < 0, u ^ jnp.int32(0x7FFFFFFF), u)
    keyp = (key & jnp.int32(~_IDX_MASK)) | (_IDX_MASK - col)
    sentinel = jnp.int32(-2147483648)
    m = jnp.max(keyp, axis=1)
    for t in range(_K):
        if t > 0:
            m = jnp.max(jnp.where(keyp < m[:, None], keyp, sentinel), axis=1)
        ti_ref[:, t] = _IDX_MASK - (m & _IDX_MASK)
        kv = (m & jnp.int32(~_IDX_MASK)) | jnp.int32(0x800)
        uv = jnp.where(kv < 0, kv ^ jnp.int32(0x7FFFFFFF), kv)
        tv_ref[:, t] = jax.lax.bitcast_convert_type(uv, jnp.float32)


def _scale_kernel(tv_ref, preds_ref, d_ref, ps_ref):
    rowsum = jnp.sum(tv_ref[...], axis=1, keepdims=True) + 1.0
    d = jax.lax.rsqrt(rowsum)
    d = jnp.where(jnp.isinf(d), 0.0, d)
    d_ref[...] = d
    ps_ref[:, :_C] = preds_ref[...] * d
    ps_ref[:, _C:] = jnp.zeros((_B, _CP - _C), jnp.float32)


def _sc_gather_prop(ti, tv, ps):
    mesh = plsc.VectorSubcoreMesh(core_axis_name="c", subcore_axis_name="s")

    @pl.kernel(
        out_type=jax.ShapeDtypeStruct((_B, _CP), jnp.float32),
        mesh=mesh,
        scratch_types=[
            pltpu.VMEM((_RPW, _K), jnp.int32),
            pltpu.VMEM((_RPW, _K), jnp.float32),
            pltpu.VMEM((_K, _CP), jnp.float32),
            pltpu.VMEM((1, _CP), jnp.float32),
            pltpu.SemaphoreType.DMA,
            pltpu.SemaphoreType.DMA,
        ],
    )
    def body(ti_hbm, tv_hbm, ps_hbm, out_hbm, idx_v, w_v, g_v, o_v, s1, s2):
        c = jax.lax.axis_index("c")
        s = jax.lax.axis_index("s")
        base = (c * 16 + s) * _RPW
        pltpu.async_copy(ti_hbm.at[pl.ds(base, _RPW)], idx_v, s1).wait()
        pltpu.async_copy(tv_hbm.at[pl.ds(base, _RPW)], w_v, s1).wait()

        @pl.loop(0, _RPW)
        def _(r):
            pltpu.async_copy(ps_hbm.at[idx_v.at[r]], g_v, s2).wait()
            w0 = w_v[r, pl.ds(0, 16)]
            w1 = w_v[r, pl.ds(16, 16)]
            ws = [w0[t] for t in range(16)] + [w1[t] for t in range(16)]

            @pl.loop(0, _CP // 16)
            def _(v):
                sl = pl.ds(v * 16, 16)
                acc = ws[0] * g_v[0, sl]
                for t in range(1, _K):
                    acc = acc + ws[t] * g_v[t, sl]
                o_v[0, sl] = acc

            pltpu.async_copy(o_v, out_hbm.at[pl.ds(base + r, 1)], s2).wait()

    return body(ti, tv, ps)


def _combine_kernel(alpha_ref, sp_ref, d_ref, pb_ref, out_ref):
    alpha = alpha_ref[0, 0]
    d = d_ref[...]
    out_ref[...] = ((1.0 - alpha) + alpha * d * d) * pb_ref[...] \
        + (alpha * d) * sp_ref[:, :_C]


@functools.partial(jax.jit, static_argnames=("interpret",))
def kernel(features, preds, current_alpha, interpret=False):
    fn = pl.pallas_call(
        _normalize_kernel,
        out_shape=jax.ShapeDtypeStruct((_B, _D), jnp.float32),
        interpret=interpret,
    )(features)

    tv, ti = pl.pallas_call(
        _simtopk_kernel,
        grid=(_B // _RB,),
        in_specs=[
            pl.BlockSpec((_RB, _D), lambda i: (i, 0)),
            pl.BlockSpec((_B, _D), lambda i: (0, 0)),
        ],
        out_specs=[
            pl.BlockSpec((_RB, _K), lambda i: (i, 0)),
            pl.BlockSpec((_RB, _K), lambda i: (i, 0)),
        ],
        out_shape=[
            jax.ShapeDtypeStruct((_B, _K), jnp.float32),
            jax.ShapeDtypeStruct((_B, _K), jnp.int32),
        ],
        interpret=interpret,
    )(fn, fn)

    d, ps = pl.pallas_call(
        _scale_kernel,
        out_shape=[
            jax.ShapeDtypeStruct((_B, 1), jnp.float32),
            jax.ShapeDtypeStruct((_B, _CP), jnp.float32),
        ],
        interpret=interpret,
    )(tv, preds)

    sp = _sc_gather_prop(ti, tv, ps)

    alpha = jnp.asarray(current_alpha, jnp.float32).reshape(1, 1)
    out = pl.pallas_call(
        _combine_kernel,
        grid=(_B // _RB,),
        in_specs=[
            pl.BlockSpec((1, 1), lambda i: (0, 0)),
            pl.BlockSpec((_RB, _CP), lambda i: (i, 0)),
            pl.BlockSpec((_RB, 1), lambda i: (i, 0)),
            pl.BlockSpec((_RB, _C), lambda i: (i, 0)),
        ],
        out_specs=pl.BlockSpec((_RB, _C), lambda i: (i, 0)),
        out_shape=jax.ShapeDtypeStruct((_B, _C), jnp.float32),
        interpret=interpret,
    )(alpha, sp, d, preds)
    return out


# speedup vs baseline: 2.9919x; 2.9919x over previous
import functools

import jax
import jax.numpy as jnp
from jax.experimental import pallas as pl

_B = 4096
_D = 1024
_C = 1000
_K = 32
_RB = 512
_IDX_BITS = 12
_IDX_MASK = (1 << _IDX_BITS) - 1


def _normalize_kernel(f_ref, out_ref):
    f = f_ref[...]
    n2 = jnp.sum(f * f, axis=1, keepdims=True)
    out_ref[...] = f * jax.lax.rsqrt(jnp.maximum(n2, 1e-24))


def _simselect_kernel(fb_ref, fn_ref, a_ref, rs_ref):
    sim = jax.lax.dot_general(
        fb_ref[...], fn_ref[...], (((1,), (1,)), ((), ())),
        preferred_element_type=jnp.float32)
    col = jax.lax.broadcasted_iota(jnp.int32, sim.shape, 1)
    u = jax.lax.bitcast_convert_type(sim, jnp.int32)
    key = jnp.where(u < 0, u ^ jnp.int32(0x7FFFFFFF), u)
    keyp = (key & jnp.int32(~_IDX_MASK)) | (_IDX_MASK - col)
    khi = (keyp >> 16).astype(jnp.int16)
    klo = (keyp & jnp.int32(0xFFFF)).astype(jnp.int16) ^ jnp.int16(-32768)
    one16 = jnp.int16(1)
    zero16 = jnp.int16(0)

    def count(mask):
        x = jnp.where(mask, one16, zero16)
        w = x.shape[1]
        while w > 256:
            w //= 2
            x = x[:, :w] + x[:, w:]
        return jnp.sum(x.astype(jnp.int32), axis=1,
                       keepdims=True).astype(jnp.int16)

    cnt0 = count(khi >= 0)
    k16 = jnp.int16(_K)
    thi = jnp.where(cnt0 >= k16, jnp.int16(0), jnp.int16(-16384))
    for b in range(13, -1, -1):
        tc = thi | jnp.int16(1 << b)
        thi = jnp.where(count(khi >= tc) >= k16, tc, thi)
    bnd = khi == thi
    need = k16 - count(khi > thi)
    cnt0l = count(bnd & (klo >= 0))
    tlo = jnp.where(cnt0l >= need, jnp.int16(0), jnp.int16(-32768))
    for b in range(14, -1, -1):
        tc = tlo | jnp.int16(1 << b)
        tlo = jnp.where(count(bnd & (klo >= tc)) >= need, tc, tlo)
    sel = (khi > thi) | (bnd & (klo >= tlo))
    a_f = jnp.where(sel, sim, 0.0)
    rs_ref[...] = jnp.sum(a_f, axis=1, keepdims=True)
    a_ref[...] = a_f.astype(jnp.bfloat16)


def _scale_kernel(rs_ref, preds_ref, d_ref, ps_ref):
    d = jax.lax.rsqrt(rs_ref[...] + 1.0)
    d = jnp.where(jnp.isinf(d), 0.0, d)
    d_ref[...] = d
    ps_ref[...] = (preds_ref[...] * d).astype(jnp.bfloat16)


def _prop_kernel(alpha_ref, a_ref, d_ref, pb_ref, ps_ref, out_ref):
    sp = jax.lax.dot_general(
        a_ref[...], ps_ref[...], (((1,), (0,)), ((), ())),
        preferred_element_type=jnp.float32)
    alpha = alpha_ref[0, 0]
    d = d_ref[...]
    out_ref[...] = ((1.0 - alpha) + alpha * d * d) * pb_ref[...] \
        + (alpha * d) * sp


@functools.partial(jax.jit, static_argnames=("interpret",))
def kernel(features, preds, current_alpha, interpret=False):
    fn = pl.pallas_call(
        _normalize_kernel,
        out_shape=jax.ShapeDtypeStruct((_B, _D), jnp.float32),
        interpret=interpret,
    )(features)

    adj, rs = pl.pallas_call(
        _simselect_kernel,
        grid=(_B // _RB,),
        in_specs=[
            pl.BlockSpec((_RB, _D), lambda i: (i, 0)),
            pl.BlockSpec((_B, _D), lambda i: (0, 0)),
        ],
        out_specs=[
            pl.BlockSpec((_RB, _B), lambda i: (i, 0)),
            pl.BlockSpec((_RB, 1), lambda i: (i, 0)),
        ],
        out_shape=[
            jax.ShapeDtypeStruct((_B, _B), jnp.bfloat16),
            jax.ShapeDtypeStruct((_B, 1), jnp.float32),
        ],
        interpret=interpret,
    )(fn, fn)

    d, ps = pl.pallas_call(
        _scale_kernel,
        out_shape=[
            jax.ShapeDtypeStruct((_B, 1), jnp.float32),
            jax.ShapeDtypeStruct((_B, _C), jnp.bfloat16),
        ],
        interpret=interpret,
    )(rs, preds)

    alpha = jnp.asarray(current_alpha, jnp.float32).reshape(1, 1)
    out = pl.pallas_call(
        _prop_kernel,
        grid=(_B // _RB,),
        in_specs=[
            pl.BlockSpec((1, 1), lambda i: (0, 0)),
            pl.BlockSpec((_RB, _B), lambda i: (i, 0)),
            pl.BlockSpec((_RB, 1), lambda i: (i, 0)),
            pl.BlockSpec((_RB, _C), lambda i: (i, 0)),
            pl.BlockSpec((_B, _C), lambda i: (0, 0)),
        ],
        out_specs=pl.BlockSpec((_RB, _C), lambda i: (i, 0)),
        out_shape=jax.ShapeDtypeStruct((_B, _C), jnp.float32),
        interpret=interpret,
    )(alpha, adj, d, preds, ps)
    return out


# speedup vs baseline: 3.1263x; 1.0449x over previous
import functools

import jax
import jax.numpy as jnp
from jax.experimental import pallas as pl

_B = 4096
_D = 1024
_C = 1000
_K = 32
_RB = 256
_IDX_BITS = 12
_IDX_MASK = (1 << _IDX_BITS) - 1


def _normalize_kernel(f_ref, out_ref):
    f = f_ref[...]
    n2 = jnp.sum(f * f, axis=1, keepdims=True)
    out_ref[...] = f * jax.lax.rsqrt(jnp.maximum(n2, 1e-24))


def _simselect_kernel(fb_ref, fn_ref, a_ref, rs_ref):
    sim = jax.lax.dot_general(
        fb_ref[...], fn_ref[...], (((1,), (1,)), ((), ())),
        preferred_element_type=jnp.float32)
    col = jax.lax.broadcasted_iota(jnp.int32, sim.shape, 1)
    u = jax.lax.bitcast_convert_type(sim, jnp.int32)
    key = jnp.where(u < 0, u ^ jnp.int32(0x7FFFFFFF), u)
    keyp = (key & jnp.int32(~_IDX_MASK)) | (_IDX_MASK - col)
    khi = (keyp >> 16).astype(jnp.int16)
    klo = (keyp & jnp.int32(0xFFFF)).astype(jnp.int16) ^ jnp.int16(-32768)
    one16 = jnp.int16(1)
    zero16 = jnp.int16(0)

    def count(mask):
        x = jnp.where(mask, one16, zero16)
        w = x.shape[1]
        while w > 256:
            w //= 2
            x = x[:, :w] + x[:, w:]
        return jnp.sum(x.astype(jnp.int32), axis=1,
                       keepdims=True).astype(jnp.int16)

    cnt0 = count(khi >= 0)
    k16 = jnp.int16(_K)
    thi = jnp.where(cnt0 >= k16, jnp.int16(0), jnp.int16(-16384))
    for b in range(13, -1, -1):
        tc = thi | jnp.int16(1 << b)
        thi = jnp.where(count(khi >= tc) >= k16, tc, thi)
    bnd = khi == thi
    need = k16 - count(khi > thi)
    cnt0l = count(bnd & (klo >= 0))
    tlo = jnp.where(cnt0l >= need, jnp.int16(0), jnp.int16(-32768))
    for b in range(14, -1, -1):
        tc = tlo | jnp.int16(1 << b)
        tlo = jnp.where(count(bnd & (klo >= tc)) >= need, tc, tlo)
    sel = (khi > thi) | (bnd & (klo >= tlo))
    a_f = jnp.where(sel, sim, 0.0)
    rs_ref[...] = jnp.sum(a_f, axis=1, keepdims=True)
    a_ref[...] = a_f.astype(jnp.bfloat16)


def _scale_kernel(rs_ref, preds_ref, d_ref, ps_ref):
    d = jax.lax.rsqrt(rs_ref[...] + 1.0)
    d = jnp.where(jnp.isinf(d), 0.0, d)
    d_ref[...] = d
    ps_ref[...] = (preds_ref[...] * d).astype(jnp.bfloat16)


def _prop_kernel(alpha_ref, a_ref, d_ref, pb_ref, ps_ref, out_ref):
    sp = jax.lax.dot_general(
        a_ref[...], ps_ref[...], (((1,), (0,)), ((), ())),
        preferred_element_type=jnp.float32)
    alpha = alpha_ref[0, 0]
    d = d_ref[...]
    out_ref[...] = ((1.0 - alpha) + alpha * d * d) * pb_ref[...] \
        + (alpha * d) * sp


@functools.partial(jax.jit, static_argnames=("interpret",))
def kernel(features, preds, current_alpha, interpret=False):
    fn = pl.pallas_call(
        _normalize_kernel,
        out_shape=jax.ShapeDtypeStruct((_B, _D), jnp.float32),
        interpret=interpret,
    )(features)

    adj, rs = pl.pallas_call(
        _simselect_kernel,
        grid=(_B // _RB,),
        in_specs=[
            pl.BlockSpec((_RB, _D), lambda i: (i, 0)),
            pl.BlockSpec((_B, _D), lambda i: (0, 0)),
        ],
        out_specs=[
            pl.BlockSpec((_RB, _B), lambda i: (i, 0)),
            pl.BlockSpec((_RB, 1), lambda i: (i, 0)),
        ],
        out_shape=[
            jax.ShapeDtypeStruct((_B, _B), jnp.bfloat16),
            jax.ShapeDtypeStruct((_B, 1), jnp.float32),
        ],
        interpret=interpret,
    )(fn, fn)

    d, ps = pl.pallas_call(
        _scale_kernel,
        out_shape=[
            jax.ShapeDtypeStruct((_B, 1), jnp.float32),
            jax.ShapeDtypeStruct((_B, _C), jnp.bfloat16),
        ],
        interpret=interpret,
    )(rs, preds)

    alpha = jnp.asarray(current_alpha, jnp.float32).reshape(1, 1)
    out = pl.pallas_call(
        _prop_kernel,
        grid=(_B // _RB,),
        in_specs=[
            pl.BlockSpec((1, 1), lambda i: (0, 0)),
            pl.BlockSpec((_RB, _B), lambda i: (i, 0)),
            pl.BlockSpec((_RB, 1), lambda i: (i, 0)),
            pl.BlockSpec((_RB, _C), lambda i: (i, 0)),
            pl.BlockSpec((_B, _C), lambda i: (0, 0)),
        ],
        out_specs=pl.BlockSpec((_RB, _C), lambda i: (i, 0)),
        out_shape=jax.ShapeDtypeStruct((_B, _C), jnp.float32),
        interpret=interpret,
    )(alpha, adj, d, preds, ps)
    return out
